# baseline (speedup 1.0000x reference)
import functools

import jax
import jax.numpy as jnp
from jax import lax
from jax.experimental import pallas as pl
from jax.experimental.pallas import tpu as pltpu
from jax.experimental.pallas import tpu_sc as plsc

PW = 4
NC, NS = 2, 16
NW = NC * NS
L = 16


def _allreduce16(v):
    iota = lax.iota(jnp.int32, L)
    dn = lax.GatherDimensionNumbers(offset_dims=(),
                                    collapsed_slice_dims=(0,),
                                    start_index_map=(0,))
    for sh in (8, 4, 2, 1):
        idx = jnp.bitwise_and(iota + sh, L - 1)
        g = lax.gather(v, idx[:, None], dn, slice_sizes=(1,),
                       mode=lax.GatherScatterMode.PROMISE_IN_BOUNDS)
        v = v + g
    return v


def _sqrt16(x):
    scale = jnp.full((L,), 1.0, jnp.float32)
    xr = x
    for e2 in (32, 16, 8, 4, 2, 1):
        up = jnp.float32(4.0 ** e2)
        dn = jnp.float32(4.0 ** (-e2))
        big = xr >= up
        xr = jnp.where(big, xr * dn, xr)
        scale = jnp.where(big, scale * jnp.float32(2.0 ** e2), scale)
        small = xr < dn
        xr = jnp.where(small, xr * up, xr)
        scale = jnp.where(small, scale * jnp.float32(2.0 ** (-e2)), scale)
    s = jnp.full((L,), 1.5, jnp.float32)
    for _ in range(5):
        s = 0.5 * (s + xr / s)
    return s * scale


def _sc_body(cdim, n_chunk_rows, cpw,
             table, idxt, out, idx_v, rows_v, sem_g0, sem_g1, sem_w):
    cid = lax.axis_index("c")
    sid = lax.axis_index("s")
    wid = sid * NC + cid
    ppc = n_chunk_rows // (PW * PW)
    ncc = cdim // L
    sem_g = (sem_g0, sem_g1)

    def start_chunk(c, buf):
        mloc = wid * cpw + c
        pltpu.sync_copy(idxt.at[mloc], idx_v.at[buf])
        return pltpu.async_copy(table.at[idx_v.at[buf]], rows_v.at[buf],
                                sem_g[buf])

    gh = [None, None]
    wh = [None, None]
    gh[0] = start_chunk(0, 0)
    for c in range(cpw):
        buf = c % 2
        nbuf = (c + 1) % 2
        if c + 1 < cpw:
            if wh[nbuf] is not None:
                for h in wh[nbuf]:
                    h.wait()
                wh[nbuf] = None
            gh[nbuf] = start_chunk(c + 1, nbuf)
        gh[buf].wait()
        def patch_body(pi, carry2, _buf=buf):
            def sumsq_row(i, acc):
                rw = i * ppc + pi
                for cc in range(ncc):
                    v = rows_v[_buf, rw, pl.ds(cc * L, L)]
                    acc = acc + v * v
                return acc
            acc = lax.fori_loop(0, PW * PW, sumsq_row,
                                jnp.zeros((L,), jnp.float32))
            ss = _allreduce16(acc)
            inv = 1.0 / (_sqrt16(ss) + 1e-7)

            def scale_row(i, carry3):
                rw = i * ppc + pi
                for cc in range(ncc):
                    sl = pl.ds(cc * L, L)
                    rows_v[_buf, rw, sl] = rows_v[_buf, rw, sl] * inv
                return carry3
            lax.fori_loop(0, PW * PW, scale_row, 0)
            return carry2
        lax.fori_loop(0, ppc, patch_body, 0)
        mglob = wid * cpw + c
        wh[buf] = [pltpu.async_copy(
            rows_v.at[buf].at[pl.ds(g * ppc, ppc)],
            out.at[pl.ds(mglob * ppc, ppc), pl.ds(g * cdim, cdim)],
            sem_w) for g in range(PW * PW)]
    for b in range(2):
        if wh[b] is not None:
            for h in wh[b]:
                h.wait()


def kernel(feats, num_patches, patch_ids):
    B, C, H, W = feats.shape
    P = patch_ids.shape[0]
    hw = H * W
    D = PW * PW * C

    table = jnp.transpose(feats, (0, 2, 3, 1)).reshape(B * hw, C)

    r = patch_ids[:, 0].astype(jnp.int32)
    c = patch_ids[:, 1].astype(jnp.int32)
    pos = (r * W + c).reshape(P, 1)
    k = jnp.arange(PW * PW, dtype=jnp.int32)
    offs = (k // PW) * W + (k % PW)
    idx = (jnp.arange(B, dtype=jnp.int32) * hw)[:, None, None] \
        + pos[None, :, :] + offs[None, None, :]

    total_rows = B * P * PW * PW
    n_chunk_rows = 128
    n_chunks = total_rows // n_chunk_rows
    ppc = n_chunk_rows // (PW * PW)
    idxt = idx.reshape(n_chunks, ppc, PW * PW).transpose(0, 2, 1) \
              .reshape(n_chunks, n_chunk_rows)

    mesh = plsc.VectorSubcoreMesh(core_axis_name="c", subcore_axis_name="s")
    out_ref = jax.new_ref(lax.empty((B * P, D), jnp.float32))

    cpw = n_chunks // NW
    sc_call = pl.kernel(
        functools.partial(_sc_body, C, n_chunk_rows, cpw),
        out_type=(),
        mesh=mesh,
        scratch_types=[
            pltpu.VMEM((2, n_chunk_rows), jnp.int32),
            pltpu.VMEM((2, n_chunk_rows, C), jnp.float32),
            pltpu.SemaphoreType.DMA,
            pltpu.SemaphoreType.DMA,
            pltpu.SemaphoreType.DMA,
        ],
    )
    sc_call(table, idxt, out_ref)

    out = out_ref[...]
    return (out, patch_ids)


# speedup vs baseline: 1.6705x; 1.6705x over previous
import functools

import jax
import jax.numpy as jnp
from jax import lax
from jax.experimental import pallas as pl
from jax.experimental.pallas import tpu as pltpu
from jax.experimental.pallas import tpu_sc as plsc

PW = 4
NC, NS = 2, 16
NW = NC * NS
L = 16


def _tc_body(W, P, pos_ref, x_ref, invt_ref):
    x = x_ref[...]
    hw = x.shape[0]
    rowsq = jnp.sum(x * x, axis=1, keepdims=True)
    tmp = rowsq
    for j in range(1, PW):
        tmp = tmp + jnp.roll(rowsq, -j, axis=0)
    win = tmp
    for i in range(1, PW):
        win = win + jnp.roll(tmp, -i * W, axis=0)
    lane = lax.broadcasted_iota(jnp.int32, (P, hw), 1)
    oh = jnp.where(lane == pos_ref[...], 1.0, 0.0)
    ss = lax.dot_general(oh, win, (((1,), (0,)), ((), ())),
                         preferred_element_type=jnp.float32)
    inv = 1.0 / (jnp.sqrt(ss) + 1e-7)
    invt_ref[...] = jnp.broadcast_to(inv, (P, L))


def _sc_body(cdim, n_chunk_rows, cpw, chunk_base,
             table, idxt, invt, out, idx_v, inv_v, rows_v,
             sem_g0, sem_g1, sem_w):
    cid = lax.axis_index("c")
    sid = lax.axis_index("s")
    wid = sid * NC + cid
    ppc = n_chunk_rows // (PW * PW)
    ncc = cdim // L
    sem_g = (sem_g0, sem_g1)

    def start_chunk(c, buf):
        mloc = wid * cpw + c
        pltpu.sync_copy(idxt.at[mloc], idx_v.at[buf])
        pltpu.sync_copy(invt.at[pl.ds(mloc * ppc, ppc)], inv_v.at[buf])
        return pltpu.async_copy(table.at[idx_v.at[buf]], rows_v.at[buf],
                                sem_g[buf])

    gh = [None, None]
    wh = [None, None]
    gh[0] = start_chunk(0, 0)
    for c in range(cpw):
        buf = c % 2
        nbuf = (c + 1) % 2
        if c + 1 < cpw:
            if wh[nbuf] is not None:
                for h in wh[nbuf]:
                    h.wait()
                wh[nbuf] = None
            gh[nbuf] = start_chunk(c + 1, nbuf)
        gh[buf].wait()
        for pi in range(ppc):
            inv = inv_v[buf, pi, pl.ds(0, L)]

            def scale_row(i, carry2, _buf=buf, _pi=pi, _inv=inv):
                rw = i * ppc + _pi
                for cc in range(ncc):
                    sl = pl.ds(cc * L, L)
                    rows_v[_buf, rw, sl] = rows_v[_buf, rw, sl] * _inv
                return carry2
            lax.fori_loop(0, PW * PW, scale_row, 0)
        mglob = chunk_base + wid * cpw + c
        wh[buf] = [pltpu.async_copy(
            rows_v.at[buf].at[pl.ds(g * ppc, ppc)],
            out.at[pl.ds(mglob * ppc, ppc), pl.ds(g * cdim, cdim)],
            sem_w) for g in range(PW * PW)]
    for b in range(2):
        if wh[b] is not None:
            for h in wh[b]:
                h.wait()


def kernel(feats, num_patches, patch_ids):
    B, C, H, W = feats.shape
    P = patch_ids.shape[0]
    hw = H * W
    D = PW * PW * C

    table = jnp.transpose(feats, (0, 2, 3, 1)).reshape(B * hw, C)

    r = patch_ids[:, 0].astype(jnp.int32)
    c = patch_ids[:, 1].astype(jnp.int32)
    pos = (r * W + c).reshape(P, 1)
    k = jnp.arange(PW * PW, dtype=jnp.int32)
    offs = (k // PW) * W + (k % PW)
    idx = (jnp.arange(B, dtype=jnp.int32) * hw)[:, None, None] \
        + pos[None, :, :] + offs[None, None, :]

    total_rows = B * P * PW * PW
    n_chunk_rows = 128
    n_chunks = total_rows // n_chunk_rows
    ppc = n_chunk_rows // (PW * PW)
    idxt = idx.reshape(n_chunks, ppc, PW * PW).transpose(0, 2, 1) \
              .reshape(n_chunks, n_chunk_rows)

    mesh = plsc.VectorSubcoreMesh(core_axis_name="c", subcore_axis_name="s")
    out_ref = jax.new_ref(lax.empty((B * P, D), jnp.float32))

    invt = pl.pallas_call(
        functools.partial(_tc_body, W, P),
        grid=(B,),
        in_specs=[
            pl.BlockSpec((P, 1), lambda b: (0, 0)),
            pl.BlockSpec((hw, C), lambda b: (b, 0)),
        ],
        out_specs=pl.BlockSpec((P, L), lambda b: (b, 0)),
        out_shape=jax.ShapeDtypeStruct((B * P, L), jnp.float32),
    )(pos, table)

    cpw = n_chunks // NW
    sc_call = pl.kernel(
        functools.partial(_sc_body, C, n_chunk_rows, cpw, 0),
        out_type=(),
        mesh=mesh,
        scratch_types=[
            pltpu.VMEM((2, n_chunk_rows), jnp.int32),
            pltpu.VMEM((2, ppc, L), jnp.float32),
            pltpu.VMEM((2, n_chunk_rows, C), jnp.float32),
            pltpu.SemaphoreType.DMA,
            pltpu.SemaphoreType.DMA,
            pltpu.SemaphoreType.DMA,
        ],
    )
    sc_call(table, idxt, invt, out_ref)

    out = out_ref[...]
    return (out, patch_ids)
